# baseline (speedup 1.0000x reference)
import jax
import jax.numpy as jnp
from jax.experimental import pallas as pl
from jax.experimental.pallas import tpu as pltpu

_HIDDEN = 128
_NCOLORS = 10
_KPAD = 16
_BB = 8


def _embed_block(grid_ref, tab_ref, pos_ref, out_ref):
    bb, h, w = grid_ref.shape
    wp = (w + 7) // 8 * 8
    g = grid_ref[...]
    gp = jnp.pad(g, ((0, 0), (0, 0), (0, wp - w)), constant_values=_KPAD - 1)
    oh = (gp[..., None] == jax.lax.broadcasted_iota(
        jnp.int32, (bb, h, wp, _KPAD), 3)).astype(jnp.float32)
    x = jnp.dot(oh.reshape(bb * h * wp, _KPAD), tab_ref[...],
                preferred_element_type=jnp.float32)
    x4 = x.reshape(bb, h, wp, _HIDDEN)[:, :, :w, :]
    out_ref[...] = x4 + pos_ref[...][None]


def kernel(grid, color_embed, pos_embed):
    b, h, w = grid.shape
    g = grid.astype(jnp.int32)
    tab = jnp.zeros((_KPAD, _HIDDEN), jnp.float32).at[:_NCOLORS].set(color_embed)
    pos = pos_embed[:h, :w]
    return pl.pallas_call(
        _embed_block,
        grid=(b // _BB,),
        in_specs=[
            pl.BlockSpec((_BB, h, w), lambda i: (i, 0, 0)),
            pl.BlockSpec((_KPAD, _HIDDEN), lambda i: (0, 0)),
            pl.BlockSpec((h, w, _HIDDEN), lambda i: (0, 0, 0)),
        ],
        out_specs=pl.BlockSpec((_BB, h, w, _HIDDEN), lambda i: (i, 0, 0, 0)),
        out_shape=jax.ShapeDtypeStruct((b, h, w, _HIDDEN), jnp.float32),
        compiler_params=pltpu.CompilerParams(
            dimension_semantics=("parallel",)),
    )(g, tab, pos)


# speedup vs baseline: 2.6208x; 2.6208x over previous
import jax
import jax.numpy as jnp
from jax.experimental import pallas as pl
from jax.experimental.pallas import tpu as pltpu

_HIDDEN = 128
_NCOLORS = 10
_KPAD = 16
_BB = 8


def _embed_block(grid_ref, tab_ref, pos_ref, out_ref):
    bb, h, w = grid_ref.shape
    g = jnp.transpose(grid_ref[...], (1, 2, 0))
    oh = (g[..., None] == jax.lax.broadcasted_iota(
        jnp.int32, (h, w, bb, _KPAD), 3)).astype(jnp.float32)
    x = jnp.dot(oh.reshape(h * w * bb, _KPAD), tab_ref[...],
                preferred_element_type=jnp.float32)
    out_ref[...] = x.reshape(h, w, bb, _HIDDEN) + pos_ref[...][:, :, None, :]


def kernel(grid, color_embed, pos_embed):
    b, h, w = grid.shape
    g = grid.astype(jnp.int32)
    tab = jnp.zeros((_KPAD, _HIDDEN), jnp.float32).at[:_NCOLORS].set(color_embed)
    pos = pos_embed[:h, :w]
    out = pl.pallas_call(
        _embed_block,
        grid=(b // _BB,),
        in_specs=[
            pl.BlockSpec((_BB, h, w), lambda i: (i, 0, 0)),
            pl.BlockSpec((_KPAD, _HIDDEN), lambda i: (0, 0)),
            pl.BlockSpec((h, w, _HIDDEN), lambda i: (0, 0, 0)),
        ],
        out_specs=pl.BlockSpec((h, w, _BB, _HIDDEN), lambda i: (0, 0, i, 0)),
        out_shape=jax.ShapeDtypeStruct((h, w, b, _HIDDEN), jnp.float32),
        compiler_params=pltpu.CompilerParams(
            dimension_semantics=("parallel",)),
    )(g, tab, pos)
    return jnp.transpose(out, (2, 0, 1, 3))
